# baseline (speedup 1.0000x reference)
import functools

import jax
import jax.numpy as jnp
from jax import lax
from jax.experimental import pallas as pl
from jax.experimental.pallas import tpu as pltpu
from jax.experimental.pallas import tpu_sc as plsc

K = 4
R = 2048
NC = 2
NS = 16
NW = NC * NS
ROWS_PER_W = R // NW
NB = 2
G = ROWS_PER_W // NB


def _make_sc_upsample():
    mesh = plsc.VectorSubcoreMesh(core_axis_name="c", subcore_axis_name="s")

    @functools.partial(
        pl.kernel,
        mesh=mesh,
        out_type=jax.ShapeDtypeStruct((R, K, 4, 1024), jnp.float32),
        scratch_types=[
            pltpu.VMEM((2 * NB, K, 4, 1024), jnp.float32),
            pltpu.SemaphoreType.DMA,
            pltpu.SemaphoreType.DMA,
            pltpu.SemaphoreType.DMA,
            pltpu.SemaphoreType.DMA,
        ],
        compiler_params=pltpu.CompilerParams(use_tc_tiling_on_sc=True),
    )
    def upsample(xs_hbm, out_hbm, ebuf, lsem0, lsem1, ssem0, ssem1):
        wid = lax.axis_index("s") * NC + lax.axis_index("c")
        base = wid * ROWS_PER_W
        lsems = (lsem0, lsem1)
        ssems = (ssem0, ssem1)

        loads = [None] * G
        stores = [None] * G

        def issue_loads(g):
            par = g % 2
            row0 = base + g * NB
            hs = []
            for b in range(NB):
                for r in range(K):
                    hs.append(
                        pltpu.async_copy(
                            xs_hbm.at[pl.ds(row0 + b, 1)],
                            ebuf.at[pl.ds(par * NB + b, 1), pl.ds(r, 1)],
                            lsems[par],
                        )
                    )
            return hs

        loads[0] = issue_loads(0)
        for g in range(G):
            par = g % 2
            if g + 1 < G:
                if g - 1 >= 0:
                    stores[g - 1].wait()
                loads[g + 1] = issue_loads(g + 1)
            for h in loads[g]:
                h.wait()
            stores[g] = pltpu.async_copy(
                ebuf.at[pl.ds(par * NB, NB)],
                out_hbm.at[pl.ds(base + g * NB, NB)],
                ssems[par],
            )
        stores[G - 2].wait()
        stores[G - 1].wait()

    return upsample


_sc_upsample = _make_sc_upsample()


def kernel(x, x_short):
    xs = x_short.reshape(R, 1, 4, 1024)
    out = _sc_upsample(xs)
    return out.reshape(R * K, 4, 1024)
